# baseline (speedup 1.0000x reference)
import functools

import jax
import jax.numpy as jnp
from jax import lax
from jax.experimental import pallas as pl
from jax.experimental.pallas import tpu as pltpu
from jax.experimental.pallas import tpu_sc as plsc

N_POINTS = 1048576
N_FACES = 1538

NUM_CORES = 2
NUM_SUBCORES = 16
NW = NUM_CORES * NUM_SUBCORES
PTS_PER_W = N_POINTS // NW
CHUNK = 8192
N_CHUNKS = PTS_PER_W // CHUNK
GROUPS = CHUNK // 16

_mesh = plsc.VectorSubcoreMesh(
    core_axis_name="c", subcore_axis_name="s", num_cores=NUM_CORES
)


@functools.partial(
    pl.kernel,
    out_type=jax.ShapeDtypeStruct((N_POINTS * 2,), jnp.float32),
    mesh=_mesh,
    compiler_params=pltpu.CompilerParams(needs_layout_passes=False),
    scratch_types=[
        pltpu.VMEM((N_FACES * 6,), jnp.float32),
        pltpu.VMEM((CHUNK,), jnp.int32),
        pltpu.VMEM((CHUNK * 3,), jnp.float32),
        pltpu.VMEM((CHUNK * 2,), jnp.float32),
    ],
)
def _uv_kernel(table_hbm, fid_hbm, bary_hbm, out_hbm, table_v, fid_v, bary_v, out_v):
    wid = lax.axis_index("s") * NUM_CORES + lax.axis_index("c")
    pltpu.sync_copy(table_hbm, table_v)

    lane = lax.iota(jnp.int32, 16)
    lane3 = lane * 3
    lane2 = lane * 2

    def chunk_body(ci, _):
        base = wid * PTS_PER_W + ci * CHUNK
        pltpu.sync_copy(fid_hbm.at[pl.ds(base, CHUNK)], fid_v)
        pltpu.sync_copy(bary_hbm.at[pl.ds(base * 3, CHUNK * 3)], bary_v)

        @plsc.parallel_loop(0, GROUPS, unroll=8)
        def group_body(g):
            gb = g * 16
            fid = fid_v[pl.ds(gb, 16)]
            tix = fid * 6
            t0 = plsc.load_gather(table_v, [tix])
            t1 = plsc.load_gather(table_v, [tix + 1])
            t2 = plsc.load_gather(table_v, [tix + 2])
            t3 = plsc.load_gather(table_v, [tix + 3])
            t4 = plsc.load_gather(table_v, [tix + 4])
            t5 = plsc.load_gather(table_v, [tix + 5])
            bix = gb * 3 + lane3
            b0 = plsc.load_gather(bary_v, [bix])
            b1 = plsc.load_gather(bary_v, [bix + 1])
            b2 = plsc.load_gather(bary_v, [bix + 2])
            u = t0 * b0 + t2 * b1 + t4 * b2
            v = t1 * b0 + t3 * b1 + t5 * b2
            oix = gb * 2 + lane2
            plsc.store_scatter(out_v, [oix], u)
            plsc.store_scatter(out_v, [oix + 1], v)

        pltpu.sync_copy(out_v, out_hbm.at[pl.ds(base * 2, CHUNK * 2)])
        return 0

    lax.fori_loop(0, N_CHUNKS, chunk_body, 0)


def kernel(points_bary, face_ids, faces_uvs_index):
    table = faces_uvs_index.reshape(-1)
    fid = face_ids.astype(jnp.int32)
    bary = points_bary.reshape(-1)
    out = _uv_kernel(table, fid, bary)
    return out.reshape(N_POINTS, 2)


# speedup vs baseline: 20.5383x; 20.5383x over previous
import functools

import jax
import jax.numpy as jnp
from jax import lax
from jax.experimental import pallas as pl
from jax.experimental.pallas import tpu as pltpu
from jax.experimental.pallas import tpu_sc as plsc

N_POINTS = 1048576
N_FACES = 1538

NUM_CORES = 2
NUM_SUBCORES = 16
NW = NUM_CORES * NUM_SUBCORES
PTS_PER_W = N_POINTS // NW
CHUNK = 8192
N_CHUNKS = PTS_PER_W // CHUNK
GROUPS = CHUNK // 16

_mesh = plsc.VectorSubcoreMesh(
    core_axis_name="c", subcore_axis_name="s", num_cores=NUM_CORES
)


@functools.partial(
    pl.kernel,
    out_type=jax.ShapeDtypeStruct((2 * N_POINTS,), jnp.float32),
    mesh=_mesh,
    compiler_params=pltpu.CompilerParams(needs_layout_passes=False),
    scratch_types=[
        pltpu.VMEM((N_FACES * 6,), jnp.float32),
        pltpu.VMEM((CHUNK,), jnp.int32),
        pltpu.VMEM((CHUNK,), jnp.float32),
        pltpu.VMEM((CHUNK,), jnp.float32),
        pltpu.VMEM((CHUNK,), jnp.float32),
        pltpu.VMEM((CHUNK,), jnp.float32),
        pltpu.VMEM((CHUNK,), jnp.float32),
    ],
)
def _uv_kernel(
    table_hbm, fid_hbm, bary_hbm, out_hbm, table_v, fid_v, b0_v, b1_v, b2_v, u_v, v_v
):
    wid = lax.axis_index("s") * NUM_CORES + lax.axis_index("c")
    pltpu.sync_copy(table_hbm, table_v)

    def chunk_body(ci, _):
        base = wid * PTS_PER_W + ci * CHUNK
        pltpu.sync_copy(fid_hbm.at[pl.ds(base, CHUNK)], fid_v)
        pltpu.sync_copy(bary_hbm.at[pl.ds(base, CHUNK)], b0_v)
        pltpu.sync_copy(bary_hbm.at[pl.ds(N_POINTS + base, CHUNK)], b1_v)
        pltpu.sync_copy(bary_hbm.at[pl.ds(2 * N_POINTS + base, CHUNK)], b2_v)

        @plsc.parallel_loop(0, GROUPS, unroll=8)
        def group_body(g):
            gb = g * 16
            fid = fid_v[pl.ds(gb, 16)]
            tix = fid * 6
            t0 = plsc.load_gather(table_v, [tix])
            t1 = plsc.load_gather(table_v, [tix + 1])
            t2 = plsc.load_gather(table_v, [tix + 2])
            t3 = plsc.load_gather(table_v, [tix + 3])
            t4 = plsc.load_gather(table_v, [tix + 4])
            t5 = plsc.load_gather(table_v, [tix + 5])
            b0 = b0_v[pl.ds(gb, 16)]
            b1 = b1_v[pl.ds(gb, 16)]
            b2 = b2_v[pl.ds(gb, 16)]
            u_v[pl.ds(gb, 16)] = t0 * b0 + t2 * b1 + t4 * b2
            v_v[pl.ds(gb, 16)] = t1 * b0 + t3 * b1 + t5 * b2

        pltpu.sync_copy(u_v, out_hbm.at[pl.ds(base, CHUNK)])
        pltpu.sync_copy(v_v, out_hbm.at[pl.ds(N_POINTS + base, CHUNK)])
        return 0

    lax.fori_loop(0, N_CHUNKS, chunk_body, 0)


def kernel(points_bary, face_ids, faces_uvs_index):
    table = faces_uvs_index.reshape(-1)
    fid = face_ids.astype(jnp.int32)
    bary_planes = points_bary.T.reshape(-1)
    out = _uv_kernel(table, fid, bary_planes)
    return out.reshape(2, N_POINTS).T


# speedup vs baseline: 22.8513x; 1.1126x over previous
import functools

import jax
import jax.numpy as jnp
from jax import lax
from jax.experimental import pallas as pl
from jax.experimental.pallas import tpu as pltpu
from jax.experimental.pallas import tpu_sc as plsc

N_POINTS = 1048576
N_FACES = 1538

NUM_CORES = 2
NUM_SUBCORES = 16
NW = NUM_CORES * NUM_SUBCORES
PTS_PER_W = N_POINTS // NW
CHUNK = 8192
N_CHUNKS = PTS_PER_W // CHUNK
GROUPS = CHUNK // 16

_mesh = plsc.VectorSubcoreMesh(
    core_axis_name="c", subcore_axis_name="s", num_cores=NUM_CORES
)


@functools.partial(
    pl.kernel,
    out_type=jax.ShapeDtypeStruct((2 * N_POINTS,), jnp.float32),
    mesh=_mesh,
    compiler_params=pltpu.CompilerParams(needs_layout_passes=False),
    scratch_types=[
        pltpu.VMEM((N_FACES * 6,), jnp.float32),
        pltpu.VMEM((CHUNK,), jnp.int32),
        pltpu.VMEM((CHUNK,), jnp.float32),
        pltpu.VMEM((CHUNK,), jnp.float32),
        pltpu.VMEM((CHUNK,), jnp.float32),
        pltpu.VMEM((2 * CHUNK,), jnp.float32),
    ],
)
def _uv_kernel(
    table_hbm, fid_hbm, bary_hbm, out_hbm, table_v, fid_v, b0_v, b1_v, b2_v, uv_v
):
    wid = lax.axis_index("s") * NUM_CORES + lax.axis_index("c")
    pltpu.sync_copy(table_hbm, table_v)

    def chunk_body(ci, _):
        base = wid * PTS_PER_W + ci * CHUNK
        pltpu.sync_copy(fid_hbm.at[pl.ds(base, CHUNK)], fid_v)
        pltpu.sync_copy(bary_hbm.at[pl.ds(base, CHUNK)], b0_v)
        pltpu.sync_copy(bary_hbm.at[pl.ds(N_POINTS + base, CHUNK)], b1_v)
        pltpu.sync_copy(bary_hbm.at[pl.ds(2 * N_POINTS + base, CHUNK)], b2_v)

        @plsc.parallel_loop(0, GROUPS, unroll=8)
        def group_body(g):
            gb = g * 16
            fid = fid_v[pl.ds(gb, 16)]
            tix = fid * 6
            t0 = plsc.load_gather(table_v, [tix])
            t1 = plsc.load_gather(table_v, [tix + 1])
            t2 = plsc.load_gather(table_v, [tix + 2])
            t3 = plsc.load_gather(table_v, [tix + 3])
            t4 = plsc.load_gather(table_v, [tix + 4])
            t5 = plsc.load_gather(table_v, [tix + 5])
            b0 = b0_v[pl.ds(gb, 16)]
            b1 = b1_v[pl.ds(gb, 16)]
            b2 = b2_v[pl.ds(gb, 16)]
            off_u = (g // 8) * 256 + (g % 8) * 16
            uv_v[pl.ds(off_u, 16)] = t0 * b0 + t2 * b1 + t4 * b2
            uv_v[pl.ds(off_u + 128, 16)] = t1 * b0 + t3 * b1 + t5 * b2

        pltpu.sync_copy(uv_v, out_hbm.at[pl.ds(2 * base, 2 * CHUNK)])
        return 0

    lax.fori_loop(0, N_CHUNKS, chunk_body, 0)


def kernel(points_bary, face_ids, faces_uvs_index):
    table = faces_uvs_index.reshape(-1)
    fid = face_ids.astype(jnp.int32)
    bary_planes = points_bary.T.reshape(-1)
    out = _uv_kernel(table, fid, bary_planes)
    return out.reshape(N_POINTS // 128, 2, 128).transpose(0, 2, 1).reshape(N_POINTS, 2)


# speedup vs baseline: 26.4764x; 1.1586x over previous
import functools

import jax
import jax.numpy as jnp
from jax import lax
from jax.experimental import pallas as pl
from jax.experimental.pallas import tpu as pltpu
from jax.experimental.pallas import tpu_sc as plsc

N_POINTS = 1048576
N_FACES = 1538

NUM_CORES = 2
NUM_SUBCORES = 16
NW = NUM_CORES * NUM_SUBCORES
PTS_PER_W = N_POINTS // NW
CHUNK = 8192
N_CHUNKS = PTS_PER_W // CHUNK
GROUPS = CHUNK // 16

_mesh = plsc.VectorSubcoreMesh(
    core_axis_name="c", subcore_axis_name="s", num_cores=NUM_CORES
)


@functools.partial(
    pl.kernel,
    out_type=jax.ShapeDtypeStruct((2 * N_POINTS,), jnp.float32),
    mesh=_mesh,
    compiler_params=pltpu.CompilerParams(needs_layout_passes=False),
    scratch_types=[
        pltpu.VMEM((N_FACES * 6,), jnp.float32),
        [pltpu.VMEM((CHUNK,), jnp.int32) for _ in range(2)],
        [pltpu.VMEM((CHUNK,), jnp.float32) for _ in range(2)],
        [pltpu.VMEM((CHUNK,), jnp.float32) for _ in range(2)],
        [pltpu.VMEM((CHUNK,), jnp.float32) for _ in range(2)],
        [pltpu.VMEM((2 * CHUNK,), jnp.float32) for _ in range(2)],
        [pltpu.SemaphoreType.DMA for _ in range(2)],
        [pltpu.SemaphoreType.DMA for _ in range(2)],
    ],
)
def _uv_kernel(
    table_hbm, fid_hbm, bary_hbm, out_hbm, table_v, fid_v, b0_v, b1_v, b2_v, uv_v,
    sem_in, sem_out,
):
    wid = lax.axis_index("s") * NUM_CORES + lax.axis_index("c")
    base0 = wid * PTS_PER_W
    pltpu.sync_copy(table_hbm, table_v)

    def start_in(ci, bi):
        base = base0 + ci * CHUNK
        return [
            pltpu.async_copy(fid_hbm.at[pl.ds(base, CHUNK)], fid_v[bi], sem_in[bi]),
            pltpu.async_copy(bary_hbm.at[pl.ds(base, CHUNK)], b0_v[bi], sem_in[bi]),
            pltpu.async_copy(
                bary_hbm.at[pl.ds(N_POINTS + base, CHUNK)], b1_v[bi], sem_in[bi]
            ),
            pltpu.async_copy(
                bary_hbm.at[pl.ds(2 * N_POINTS + base, CHUNK)], b2_v[bi], sem_in[bi]
            ),
        ]

    pending_in = {0: start_in(0, 0)}
    pending_out = {}
    for ci in range(N_CHUNKS):
        bi = ci % 2
        if ci + 1 < N_CHUNKS:
            pending_in[ci + 1] = start_in(ci + 1, 1 - bi)
        for d in pending_in.pop(ci):
            d.wait()
        if ci - 2 in pending_out:
            pending_out.pop(ci - 2).wait()

        fid_b, b0_b, b1_b, b2_b, uv_b = (
            fid_v[bi], b0_v[bi], b1_v[bi], b2_v[bi], uv_v[bi]
        )

        @plsc.parallel_loop(0, GROUPS, unroll=8)
        def group_body(g):
            gb = g * 16
            fid = fid_b[pl.ds(gb, 16)]
            tix = fid * 6
            t0 = plsc.load_gather(table_v, [tix])
            t1 = plsc.load_gather(table_v, [tix + 1])
            t2 = plsc.load_gather(table_v, [tix + 2])
            t3 = plsc.load_gather(table_v, [tix + 3])
            t4 = plsc.load_gather(table_v, [tix + 4])
            t5 = plsc.load_gather(table_v, [tix + 5])
            b0 = b0_b[pl.ds(gb, 16)]
            b1 = b1_b[pl.ds(gb, 16)]
            b2 = b2_b[pl.ds(gb, 16)]
            off_u = (g // 8) * 256 + (g % 8) * 16
            uv_b[pl.ds(off_u, 16)] = t0 * b0 + t2 * b1 + t4 * b2
            uv_b[pl.ds(off_u + 128, 16)] = t1 * b0 + t3 * b1 + t5 * b2

        base = base0 + ci * CHUNK
        pending_out[ci] = pltpu.async_copy(
            uv_v[bi], out_hbm.at[pl.ds(2 * base, 2 * CHUNK)], sem_out[bi]
        )
    for d in pending_out.values():
        d.wait()


def kernel(points_bary, face_ids, faces_uvs_index):
    table = faces_uvs_index.reshape(-1)
    fid = face_ids.astype(jnp.int32)
    bary_planes = points_bary.T.reshape(-1)
    out = _uv_kernel(table, fid, bary_planes)
    return out.reshape(N_POINTS // 128, 2, 128).transpose(0, 2, 1).reshape(N_POINTS, 2)


# speedup vs baseline: 34.0807x; 1.2872x over previous
import functools

import jax
import jax.numpy as jnp
from jax import lax
from jax.experimental import pallas as pl
from jax.experimental.pallas import tpu as pltpu
from jax.experimental.pallas import tpu_sc as plsc

N_POINTS = 1048576
N_FACES = 1538

NUM_CORES = 2
NUM_SUBCORES = 16
NW = NUM_CORES * NUM_SUBCORES
PTS_PER_W = N_POINTS // NW
CHUNK = 8192
N_CHUNKS = PTS_PER_W // CHUNK
GROUPS = CHUNK // 16

_mesh = plsc.VectorSubcoreMesh(
    core_axis_name="c", subcore_axis_name="s", num_cores=NUM_CORES
)


@functools.partial(
    pl.kernel,
    out_type=jax.ShapeDtypeStruct((2 * N_POINTS,), jnp.float32),
    mesh=_mesh,
    compiler_params=pltpu.CompilerParams(needs_layout_passes=False),
    scratch_types=[
        pltpu.VMEM((N_FACES * 6,), jnp.float32),
        [pltpu.VMEM((CHUNK,), jnp.int32) for _ in range(2)],
        [pltpu.VMEM((3 * CHUNK,), jnp.float32) for _ in range(2)],
        [pltpu.VMEM((2 * CHUNK,), jnp.float32) for _ in range(2)],
        [pltpu.SemaphoreType.DMA for _ in range(2)],
        [pltpu.SemaphoreType.DMA for _ in range(2)],
    ],
)
def _uv_kernel(
    table_hbm, fid_hbm, bary_hbm, out_hbm, table_v, fid_v, bary_v, uv_v,
    sem_in, sem_out,
):
    wid = lax.axis_index("s") * NUM_CORES + lax.axis_index("c")
    base0 = wid * PTS_PER_W

    def start_in(ci, bi):
        base = base0 + ci * CHUNK
        return [
            pltpu.async_copy(fid_hbm.at[pl.ds(base, CHUNK)], fid_v[bi], sem_in[bi]),
            pltpu.async_copy(
                bary_hbm.at[pl.ds(3 * base, 3 * CHUNK)], bary_v[bi], sem_in[bi]
            ),
        ]

    pending_in = {0: start_in(0, 0)}
    pltpu.sync_copy(table_hbm, table_v)
    pending_out = {}
    for ci in range(N_CHUNKS):
        bi = ci % 2
        if ci + 1 < N_CHUNKS:
            pending_in[ci + 1] = start_in(ci + 1, 1 - bi)
        for d in pending_in.pop(ci):
            d.wait()
        if ci - 2 in pending_out:
            pending_out.pop(ci - 2).wait()

        bary_b, fid_b, uv_b = bary_v[bi], fid_v[bi], uv_v[bi]

        @plsc.parallel_loop(0, GROUPS, unroll=8)
        def group_body(g):
            fid = fid_b[pl.ds(g * 16, 16)]
            tix = fid * 6
            t0 = plsc.load_gather(table_v, [tix])
            t1 = plsc.load_gather(table_v, [tix + 1])
            t2 = plsc.load_gather(table_v, [tix + 2])
            t3 = plsc.load_gather(table_v, [tix + 3])
            t4 = plsc.load_gather(table_v, [tix + 4])
            t5 = plsc.load_gather(table_v, [tix + 5])
            blk, r = g // 8, (g % 8) * 16
            off_b = blk * 384 + r
            b0 = bary_b[pl.ds(off_b, 16)]
            b1 = bary_b[pl.ds(off_b + 128, 16)]
            b2 = bary_b[pl.ds(off_b + 256, 16)]
            off_u = blk * 256 + r
            uv_b[pl.ds(off_u, 16)] = t0 * b0 + t2 * b1 + t4 * b2
            uv_b[pl.ds(off_u + 128, 16)] = t1 * b0 + t3 * b1 + t5 * b2

        base = base0 + ci * CHUNK
        pending_out[ci] = pltpu.async_copy(
            uv_v[bi], out_hbm.at[pl.ds(2 * base, 2 * CHUNK)], sem_out[bi]
        )
    for d in pending_out.values():
        d.wait()


def kernel(points_bary, face_ids, faces_uvs_index):
    table = faces_uvs_index.reshape(-1)
    fid = face_ids.astype(jnp.int32)
    bary_blocks = (
        points_bary.T.reshape(3, N_POINTS // 128, 128)
        .transpose(1, 0, 2)
        .reshape(-1)
    )
    out = _uv_kernel(table, fid, bary_blocks)
    return out.reshape(N_POINTS // 128, 2, 128).transpose(0, 2, 1).reshape(N_POINTS, 2)
